# baseline (speedup 1.0000x reference)
import functools

import jax
import jax.numpy as jnp
import numpy as np
from jax import lax
from jax.experimental import pallas as pl
from jax.experimental.pallas import tpu as pltpu
from jax.experimental.pallas import tpu_sc as plsc

_F = 26
_E = 4
_V = 100000
_D = 16
_B = 4096

_ROWS = _B * _E * _F
_NW = 32
_RPW = _ROWS // _NW
_CHUNK = 1664
_NCHUNK = _RPW // _CHUNK
_L = 16

_pat = ((np.arange(_F)[None, :] * _E + np.arange(_E)[:, None]) * _V)
_OFFSETS = np.tile(_pat.reshape(-1), _RPW // (_E * _F)).astype(np.int32)


def _sc_body(tab_hbm, x_hbm, off_hbm, out_hbm, idx_v, off_v, buf, sem):
    wid = lax.axis_index("s") * 2 + lax.axis_index("c")
    base = wid * _RPW
    pltpu.sync_copy(x_hbm.at[pl.ds(base, _RPW)], idx_v)
    pltpu.sync_copy(off_hbm, off_v)

    def _add(i, _):
        s = pl.ds(i * _L, _L)
        idx_v[s] = idx_v[s] + off_v[s]
        return 0

    lax.fori_loop(0, _RPW // _L, _add, 0)

    for c in range(_NCHUNK):
        pltpu.async_copy(
            tab_hbm.at[idx_v.at[pl.ds(c * _CHUNK, _CHUNK)]], buf, sem
        ).wait()
        pltpu.sync_copy(buf, out_hbm.at[pl.ds(base + c * _CHUNK, _CHUNK)])


@jax.jit
def kernel(x, tables):
    tab_flat = tables.reshape(_F * _E * _V, _D)
    x_flat = x.reshape(_ROWS)
    mesh = plsc.VectorSubcoreMesh(core_axis_name="c", subcore_axis_name="s")
    run = pl.kernel(
        _sc_body,
        mesh=mesh,
        out_type=jax.ShapeDtypeStruct((_ROWS, _D), jnp.float32),
        scratch_types=[
            pltpu.VMEM((_RPW,), jnp.int32),
            pltpu.VMEM((_RPW,), jnp.int32),
            pltpu.VMEM((_CHUNK, _D), jnp.float32),
            pltpu.SemaphoreType.DMA,
        ],
        compiler_params=pltpu.CompilerParams(use_tc_tiling_on_sc=False),
    )
    out = run(tab_flat, x_flat, jnp.asarray(_OFFSETS))
    return out.reshape(_B, _E, _F * _D)
